# baseline (speedup 1.0000x reference)
import functools

import jax
import jax.numpy as jnp
from jax import lax
from jax.experimental import pallas as pl
from jax.experimental.pallas import tpu as pltpu
from jax.experimental.pallas import tpu_sc as plsc

N_NODES = 10000
D_FEAT = 256
EMB_DIM = 256
N_CLASSES = 40
N_EDGES = 160000

NC, NS = 2, 16
NW = NC * NS
CHUNK = 128
NP = 10112
EP = NW * 40 * CHUNK
ROWS_PER_TILE = NP // NS

_MESH = plsc.VectorSubcoreMesh(
    core_axis_name="c", subcore_axis_name="s", num_cores=NC, num_subcores=NS)


@functools.partial(
    pl.kernel,
    out_type=jax.ShapeDtypeStruct((NC, NP), jnp.float32),
    mesh=_MESH,
    scratch_types=[
        pltpu.VMEM((40 * CHUNK,), jnp.int32),
        pltpu.VMEM((40 * CHUNK,), jnp.float32),
        pltpu.VMEM_SHARED((NP,), jnp.float32),
    ],
)
def _deg_kernel(dst_hbm, zer_hbm, one_hbm, out_hbm, dst_v, one_v, acc):
    c = lax.axis_index("c")
    s = lax.axis_index("s")
    w = c * NS + s
    pltpu.sync_copy(dst_hbm.at[pl.ds(w * 40 * CHUNK, 40 * CHUNK)], dst_v)
    pltpu.sync_copy(one_hbm, one_v)

    @pl.when(s == 0)
    def _():
        pltpu.sync_copy(zer_hbm, acc)

    plsc.subcore_barrier()
    pltpu.sync_copy(one_v, acc.at[dst_v], add=True)
    plsc.subcore_barrier()

    @pl.when(s == 0)
    def _():
        pltpu.sync_copy(acc, out_hbm.at[c])


@functools.partial(
    pl.kernel,
    out_type=jax.ShapeDtypeStruct((NC, NP, 128), jnp.float32),
    mesh=_MESH,
    scratch_types=[
        pltpu.VMEM((40, CHUNK), jnp.int32),
        pltpu.VMEM((40, CHUNK), jnp.int32),
        pltpu.VMEM((2, CHUNK, 128), jnp.float32),
        pltpu.VMEM_SHARED((NP, 128), jnp.float32),
        pltpu.SemaphoreType.DMA,
        pltpu.SemaphoreType.DMA,
        pltpu.SemaphoreType.DMA,
        pltpu.SemaphoreType.DMA,
    ],
)
def _agg256_kernel(src_hbm, dst_hbm, h_hbm, zer_hbm, out_hbm,
                   src_v, dst_v, gbuf, acc, gs0, gs1, ss0, ss1):
    c = lax.axis_index("c")
    s = lax.axis_index("s")
    h_mine = h_hbm.at[c]
    pltpu.sync_copy(zer_hbm, acc.at[pl.ds(s * ROWS_PER_TILE, ROWS_PER_TILE)])
    plsc.subcore_barrier()

    for half in range(2):
        base = s * 80 + half * 40
        pltpu.sync_copy(src_hbm.at[pl.ds(base, 40)], src_v)
        pltpu.sync_copy(dst_hbm.at[pl.ds(base, 40)], dst_v)

        def pair(j, _):
            @pl.when(j > 0)
            def _():
                pltpu.make_async_copy(gbuf.at[0], acc.at[dst_v.at[0]], ss0).wait()
            pltpu.async_copy(h_mine.at[src_v.at[2 * j]], gbuf.at[0], gs0)

            @pl.when(j > 0)
            def _():
                pltpu.make_async_copy(gbuf.at[1], acc.at[dst_v.at[0]], ss1).wait()
            pltpu.async_copy(h_mine.at[src_v.at[2 * j + 1]], gbuf.at[1], gs1)

            pltpu.make_async_copy(h_mine.at[src_v.at[0]], gbuf.at[0], gs0).wait()
            pltpu.async_copy(gbuf.at[0], acc.at[dst_v.at[2 * j]], ss0, add=True)
            pltpu.make_async_copy(h_mine.at[src_v.at[0]], gbuf.at[1], gs1).wait()
            pltpu.async_copy(gbuf.at[1], acc.at[dst_v.at[2 * j + 1]], ss1, add=True)
            return ()

        lax.fori_loop(0, 20, pair, (), unroll=False)
        pltpu.make_async_copy(gbuf.at[0], acc.at[dst_v.at[0]], ss0).wait()
        pltpu.make_async_copy(gbuf.at[1], acc.at[dst_v.at[0]], ss1).wait()
    plsc.subcore_barrier()
    pltpu.sync_copy(acc.at[pl.ds(s * ROWS_PER_TILE, ROWS_PER_TILE)],
                    out_hbm.at[c].at[pl.ds(s * ROWS_PER_TILE, ROWS_PER_TILE)])


@functools.partial(
    pl.kernel,
    out_type=jax.ShapeDtypeStruct((NC, NP, 128), jnp.float32),
    mesh=_MESH,
    scratch_types=[
        pltpu.VMEM((40, CHUNK), jnp.int32),
        pltpu.VMEM((40, CHUNK), jnp.int32),
        pltpu.VMEM((2, CHUNK, 128), jnp.float32),
        pltpu.VMEM_SHARED((NP, 128), jnp.float32),
        pltpu.SemaphoreType.DMA,
        pltpu.SemaphoreType.DMA,
        pltpu.SemaphoreType.DMA,
        pltpu.SemaphoreType.DMA,
    ],
)
def _agg40_kernel(src_hbm, dst_hbm, h_hbm, zer_hbm, out_hbm,
                  src_v, dst_v, gbuf, acc, gs0, gs1, ss0, ss1):
    c = lax.axis_index("c")
    s = lax.axis_index("s")
    w = c * NS + s
    pltpu.sync_copy(src_hbm.at[pl.ds(w * 40, 40)], src_v)
    pltpu.sync_copy(dst_hbm.at[pl.ds(w * 40, 40)], dst_v)
    pltpu.sync_copy(zer_hbm,
                    acc.at[pl.ds(s * ROWS_PER_TILE, ROWS_PER_TILE)])
    plsc.subcore_barrier()

    def pair(j, _):
        @pl.when(j > 0)
        def _():
            pltpu.make_async_copy(gbuf.at[0], acc.at[dst_v.at[0]], ss0).wait()
        pltpu.async_copy(h_hbm.at[src_v.at[2 * j]], gbuf.at[0], gs0)

        @pl.when(j > 0)
        def _():
            pltpu.make_async_copy(gbuf.at[1], acc.at[dst_v.at[0]], ss1).wait()
        pltpu.async_copy(h_hbm.at[src_v.at[2 * j + 1]], gbuf.at[1], gs1)

        pltpu.make_async_copy(h_hbm.at[src_v.at[0]], gbuf.at[0], gs0).wait()
        pltpu.async_copy(gbuf.at[0], acc.at[dst_v.at[2 * j]], ss0, add=True)
        pltpu.make_async_copy(h_hbm.at[src_v.at[0]], gbuf.at[1], gs1).wait()
        pltpu.async_copy(gbuf.at[1], acc.at[dst_v.at[2 * j + 1]], ss1, add=True)
        return ()

    lax.fori_loop(0, 20, pair, (), unroll=False)
    pltpu.make_async_copy(gbuf.at[0], acc.at[dst_v.at[0]], ss0).wait()
    pltpu.make_async_copy(gbuf.at[1], acc.at[dst_v.at[0]], ss1).wait()
    plsc.subcore_barrier()
    pltpu.sync_copy(acc.at[pl.ds(s * ROWS_PER_TILE, ROWS_PER_TILE)],
                    out_hbm.at[c].at[pl.ds(s * ROWS_PER_TILE, ROWS_PER_TILE)])


def _k2_body(x_ref, w1_ref, deg_ref, h1p_ref, dis_ref):
    deg = deg_ref[0] + deg_ref[1]
    dis = lax.rsqrt(deg + 1.0)
    h = jnp.dot(x_ref[...], w1_ref[...], preferred_element_type=jnp.float32)
    hp = h * dis
    h1p_ref[0] = hp[:, :128]
    h1p_ref[1] = hp[:, 128:]
    dis_ref[...] = dis


def _k4_body(agg_ref, h1p_ref, dis_ref, b1_ref, w2_ref, h2p_ref):
    dis = dis_ref[...]
    z0 = dis * (agg_ref[0] + h1p_ref[0])
    z1 = dis * (agg_ref[1] + h1p_ref[1])
    z = jnp.concatenate([z0, z1], axis=1) + b1_ref[...]
    z = jnp.maximum(z, 0.0)
    h2 = jnp.dot(z, w2_ref[...], preferred_element_type=jnp.float32)
    h2p_ref[...] = jnp.pad(h2 * dis, ((0, 0), (0, 128 - N_CLASSES)))


def _k6_body(agg_ref, h2p_ref, dis_ref, b2_ref, out_ref):
    s = agg_ref[0] + agg_ref[1] + h2p_ref[...]
    out_ref[...] = dis_ref[...] * s[:, :N_CLASSES] + b2_ref[...]


_BM = 2528


def _tc_k2(x, w1, deg):
    return pl.pallas_call(
        _k2_body,
        grid=(NP // _BM,),
        in_specs=[
            pl.BlockSpec((_BM, D_FEAT), lambda i: (i, 0)),
            pl.BlockSpec((D_FEAT, EMB_DIM), lambda i: (0, 0)),
            pl.BlockSpec((NC, _BM, 1), lambda i: (0, i, 0)),
        ],
        out_specs=[
            pl.BlockSpec((NC, _BM, 128), lambda i: (0, i, 0)),
            pl.BlockSpec((_BM, 1), lambda i: (i, 0)),
        ],
        out_shape=[
            jax.ShapeDtypeStruct((NC, NP, 128), jnp.float32),
            jax.ShapeDtypeStruct((NP, 1), jnp.float32),
        ],
    )(x, w1, deg)


def _tc_k4(agg1, h1p, dis, b1, w2):
    return pl.pallas_call(
        _k4_body,
        grid=(NP // _BM,),
        in_specs=[
            pl.BlockSpec((NC, _BM, 128), lambda i: (0, i, 0)),
            pl.BlockSpec((NC, _BM, 128), lambda i: (0, i, 0)),
            pl.BlockSpec((_BM, 1), lambda i: (i, 0)),
            pl.BlockSpec((1, EMB_DIM), lambda i: (0, 0)),
            pl.BlockSpec((EMB_DIM, N_CLASSES), lambda i: (0, 0)),
        ],
        out_specs=pl.BlockSpec((_BM, 128), lambda i: (i, 0)),
        out_shape=jax.ShapeDtypeStruct((NP, 128), jnp.float32),
    )(agg1, h1p, dis, b1, w2)


def _tc_k6(agg2, h2p, dis, b2):
    return pl.pallas_call(
        _k6_body,
        grid=(NP // _BM,),
        in_specs=[
            pl.BlockSpec((NC, _BM, 128), lambda i: (0, i, 0)),
            pl.BlockSpec((_BM, 128), lambda i: (i, 0)),
            pl.BlockSpec((_BM, 1), lambda i: (i, 0)),
            pl.BlockSpec((1, N_CLASSES), lambda i: (0, 0)),
        ],
        out_specs=pl.BlockSpec((_BM, N_CLASSES), lambda i: (i, 0)),
        out_shape=jax.ShapeDtypeStruct((NP, N_CLASSES), jnp.float32),
    )(agg2, h2p, dis, b2)


def kernel(x, edge_index, W1, b1, W2, b2):
    src = edge_index[0].astype(jnp.int32)
    dst = edge_index[1].astype(jnp.int32)

    xp = jnp.concatenate(
        [x, jnp.zeros((NP - N_NODES, D_FEAT), jnp.float32)], axis=0)
    npad = EP - N_EDGES
    pad_idx = N_NODES + (jnp.arange(npad, dtype=jnp.int32) % 8)
    src_p = jnp.concatenate([src, pad_idx]).reshape(EP // CHUNK, CHUNK)
    dst_p = jnp.concatenate([dst, pad_idx]).reshape(EP // CHUNK, CHUNK)

    zeros_128 = jnp.zeros((ROWS_PER_TILE, 128), jnp.float32)
    zeros_1 = jnp.zeros((NP,), jnp.float32)
    ones_c = jnp.ones((40 * CHUNK,), jnp.float32)

    deg = _deg_kernel(dst_p.reshape(EP), zeros_1, ones_c).reshape(NC, NP, 1)
    h1p, dis = _tc_k2(xp, W1, deg)
    agg1 = _agg256_kernel(src_p, dst_p, h1p, zeros_128)
    h2p = _tc_k4(agg1, h1p, dis, b1.reshape(1, EMB_DIM), W2)
    agg2 = _agg40_kernel(src_p, dst_p, h2p, zeros_128)
    out = _tc_k6(agg2, h2p, dis, b2.reshape(1, N_CLASSES))
    return out[:N_NODES]


# speedup vs baseline: 1.0265x; 1.0265x over previous
import functools

import jax
import jax.numpy as jnp
from jax import lax
from jax.experimental import pallas as pl
from jax.experimental.pallas import tpu as pltpu
from jax.experimental.pallas import tpu_sc as plsc

N_NODES = 10000
D_FEAT = 256
EMB_DIM = 256
N_CLASSES = 40
N_EDGES = 160000

NC, NS = 2, 16
NW = NC * NS
CHUNK = 128
NP = 10112
EP = NW * 40 * CHUNK
ROWS_PER_TILE = NP // NS

_MESH = plsc.VectorSubcoreMesh(
    core_axis_name="c", subcore_axis_name="s", num_cores=NC, num_subcores=NS)


@functools.partial(
    pl.kernel,
    out_type=jax.ShapeDtypeStruct((NC, NP), jnp.float32),
    mesh=_MESH,
    scratch_types=[
        pltpu.VMEM((40, CHUNK), jnp.int32),
        pltpu.VMEM((CHUNK,), jnp.float32),
        pltpu.VMEM_SHARED((NP,), jnp.float32),
    ],
)
def _deg_kernel(dst_hbm, zer_hbm, one_hbm, out_hbm, dst_v, one_v, acc):
    c = lax.axis_index("c")
    s = lax.axis_index("s")
    w = c * NS + s
    pltpu.sync_copy(dst_hbm.at[pl.ds(w * 40, 40)], dst_v)
    pltpu.sync_copy(one_hbm, one_v)

    @pl.when(s == 0)
    def _():
        pltpu.sync_copy(zer_hbm, acc)

    plsc.subcore_barrier()

    def body(j, _):
        pltpu.sync_copy(one_v, acc.at[dst_v.at[j]], add=True)
        return ()

    lax.fori_loop(0, 40, body, (), unroll=False)
    plsc.subcore_barrier()

    @pl.when(s == 0)
    def _():
        pltpu.sync_copy(acc, out_hbm.at[c])


@functools.partial(
    pl.kernel,
    out_type=jax.ShapeDtypeStruct((NC, NP, 128), jnp.float32),
    mesh=_MESH,
    scratch_types=[
        pltpu.VMEM((40, CHUNK), jnp.int32),
        pltpu.VMEM((40, CHUNK), jnp.int32),
        pltpu.VMEM((2, CHUNK, 128), jnp.float32),
        pltpu.VMEM_SHARED((NP, 128), jnp.float32),
        pltpu.SemaphoreType.DMA,
        pltpu.SemaphoreType.DMA,
        pltpu.SemaphoreType.DMA,
        pltpu.SemaphoreType.DMA,
    ],
)
def _agg256_kernel(src_hbm, dst_hbm, h_hbm, zer_hbm, out_hbm,
                   src_v, dst_v, gbuf, acc, gs0, gs1, ss0, ss1):
    c = lax.axis_index("c")
    s = lax.axis_index("s")
    h_mine = h_hbm.at[c]
    pltpu.sync_copy(zer_hbm, acc.at[pl.ds(s * ROWS_PER_TILE, ROWS_PER_TILE)])
    plsc.subcore_barrier()

    for half in range(2):
        base = s * 80 + half * 40
        pltpu.sync_copy(src_hbm.at[pl.ds(base, 40)], src_v)
        pltpu.sync_copy(dst_hbm.at[pl.ds(base, 40)], dst_v)

        def pair(j, _):
            @pl.when(j > 0)
            def _():
                pltpu.make_async_copy(gbuf.at[0], acc.at[dst_v.at[0]], ss0).wait()
            pltpu.async_copy(h_mine.at[src_v.at[2 * j]], gbuf.at[0], gs0)

            @pl.when(j > 0)
            def _():
                pltpu.make_async_copy(gbuf.at[1], acc.at[dst_v.at[0]], ss1).wait()
            pltpu.async_copy(h_mine.at[src_v.at[2 * j + 1]], gbuf.at[1], gs1)

            pltpu.make_async_copy(h_mine.at[src_v.at[0]], gbuf.at[0], gs0).wait()
            pltpu.async_copy(gbuf.at[0], acc.at[dst_v.at[2 * j]], ss0, add=True)
            pltpu.make_async_copy(h_mine.at[src_v.at[0]], gbuf.at[1], gs1).wait()
            pltpu.async_copy(gbuf.at[1], acc.at[dst_v.at[2 * j + 1]], ss1, add=True)
            return ()

        lax.fori_loop(0, 20, pair, (), unroll=False)
        pltpu.make_async_copy(gbuf.at[0], acc.at[dst_v.at[0]], ss0).wait()
        pltpu.make_async_copy(gbuf.at[1], acc.at[dst_v.at[0]], ss1).wait()
    plsc.subcore_barrier()
    pltpu.sync_copy(acc.at[pl.ds(s * ROWS_PER_TILE, ROWS_PER_TILE)],
                    out_hbm.at[c].at[pl.ds(s * ROWS_PER_TILE, ROWS_PER_TILE)])


@functools.partial(
    pl.kernel,
    out_type=jax.ShapeDtypeStruct((NC, NP, 128), jnp.float32),
    mesh=_MESH,
    scratch_types=[
        pltpu.VMEM((40, CHUNK), jnp.int32),
        pltpu.VMEM((40, CHUNK), jnp.int32),
        pltpu.VMEM((2, CHUNK, 128), jnp.float32),
        pltpu.VMEM_SHARED((NP, 128), jnp.float32),
        pltpu.SemaphoreType.DMA,
        pltpu.SemaphoreType.DMA,
        pltpu.SemaphoreType.DMA,
        pltpu.SemaphoreType.DMA,
    ],
)
def _agg40_kernel(src_hbm, dst_hbm, h_hbm, zer_hbm, out_hbm,
                  src_v, dst_v, gbuf, acc, gs0, gs1, ss0, ss1):
    c = lax.axis_index("c")
    s = lax.axis_index("s")
    w = c * NS + s
    pltpu.sync_copy(src_hbm.at[pl.ds(w * 40, 40)], src_v)
    pltpu.sync_copy(dst_hbm.at[pl.ds(w * 40, 40)], dst_v)
    pltpu.sync_copy(zer_hbm,
                    acc.at[pl.ds(s * ROWS_PER_TILE, ROWS_PER_TILE)])
    plsc.subcore_barrier()

    def pair(j, _):
        @pl.when(j > 0)
        def _():
            pltpu.make_async_copy(gbuf.at[0], acc.at[dst_v.at[0]], ss0).wait()
        pltpu.async_copy(h_hbm.at[src_v.at[2 * j]], gbuf.at[0], gs0)

        @pl.when(j > 0)
        def _():
            pltpu.make_async_copy(gbuf.at[1], acc.at[dst_v.at[0]], ss1).wait()
        pltpu.async_copy(h_hbm.at[src_v.at[2 * j + 1]], gbuf.at[1], gs1)

        pltpu.make_async_copy(h_hbm.at[src_v.at[0]], gbuf.at[0], gs0).wait()
        pltpu.async_copy(gbuf.at[0], acc.at[dst_v.at[2 * j]], ss0, add=True)
        pltpu.make_async_copy(h_hbm.at[src_v.at[0]], gbuf.at[1], gs1).wait()
        pltpu.async_copy(gbuf.at[1], acc.at[dst_v.at[2 * j + 1]], ss1, add=True)
        return ()

    lax.fori_loop(0, 20, pair, (), unroll=False)
    pltpu.make_async_copy(gbuf.at[0], acc.at[dst_v.at[0]], ss0).wait()
    pltpu.make_async_copy(gbuf.at[1], acc.at[dst_v.at[0]], ss1).wait()
    plsc.subcore_barrier()
    pltpu.sync_copy(acc.at[pl.ds(s * ROWS_PER_TILE, ROWS_PER_TILE)],
                    out_hbm.at[c].at[pl.ds(s * ROWS_PER_TILE, ROWS_PER_TILE)])


def _k2_body(x_ref, w1_ref, deg_ref, h1p_ref, dis_ref):
    deg = deg_ref[0] + deg_ref[1]
    dis = lax.rsqrt(deg + 1.0)
    h = jnp.dot(x_ref[...], w1_ref[...], preferred_element_type=jnp.float32)
    hp = h * dis
    h1p_ref[0] = hp[:, :128]
    h1p_ref[1] = hp[:, 128:]
    dis_ref[...] = dis


def _k4_body(agg_ref, h1p_ref, dis_ref, b1_ref, w2_ref, h2p_ref):
    dis = dis_ref[...]
    z0 = dis * (agg_ref[0] + h1p_ref[0])
    z1 = dis * (agg_ref[1] + h1p_ref[1])
    z = jnp.concatenate([z0, z1], axis=1) + b1_ref[...]
    z = jnp.maximum(z, 0.0)
    h2 = jnp.dot(z, w2_ref[...], preferred_element_type=jnp.float32)
    h2p_ref[...] = jnp.pad(h2 * dis, ((0, 0), (0, 128 - N_CLASSES)))


def _k6_body(agg_ref, h2p_ref, dis_ref, b2_ref, out_ref):
    s = agg_ref[0] + agg_ref[1] + h2p_ref[...]
    out_ref[...] = dis_ref[...] * s[:, :N_CLASSES] + b2_ref[...]


_BM = 2528


def _tc_k2(x, w1, deg):
    return pl.pallas_call(
        _k2_body,
        grid=(NP // _BM,),
        in_specs=[
            pl.BlockSpec((_BM, D_FEAT), lambda i: (i, 0)),
            pl.BlockSpec((D_FEAT, EMB_DIM), lambda i: (0, 0)),
            pl.BlockSpec((NC, _BM, 1), lambda i: (0, i, 0)),
        ],
        out_specs=[
            pl.BlockSpec((NC, _BM, 128), lambda i: (0, i, 0)),
            pl.BlockSpec((_BM, 1), lambda i: (i, 0)),
        ],
        out_shape=[
            jax.ShapeDtypeStruct((NC, NP, 128), jnp.float32),
            jax.ShapeDtypeStruct((NP, 1), jnp.float32),
        ],
    )(x, w1, deg)


def _tc_k4(agg1, h1p, dis, b1, w2):
    return pl.pallas_call(
        _k4_body,
        grid=(NP // _BM,),
        in_specs=[
            pl.BlockSpec((NC, _BM, 128), lambda i: (0, i, 0)),
            pl.BlockSpec((NC, _BM, 128), lambda i: (0, i, 0)),
            pl.BlockSpec((_BM, 1), lambda i: (i, 0)),
            pl.BlockSpec((1, EMB_DIM), lambda i: (0, 0)),
            pl.BlockSpec((EMB_DIM, N_CLASSES), lambda i: (0, 0)),
        ],
        out_specs=pl.BlockSpec((_BM, 128), lambda i: (i, 0)),
        out_shape=jax.ShapeDtypeStruct((NP, 128), jnp.float32),
    )(agg1, h1p, dis, b1, w2)


def _tc_k6(agg2, h2p, dis, b2):
    return pl.pallas_call(
        _k6_body,
        grid=(NP // _BM,),
        in_specs=[
            pl.BlockSpec((NC, _BM, 128), lambda i: (0, i, 0)),
            pl.BlockSpec((_BM, 128), lambda i: (i, 0)),
            pl.BlockSpec((_BM, 1), lambda i: (i, 0)),
            pl.BlockSpec((1, N_CLASSES), lambda i: (0, 0)),
        ],
        out_specs=pl.BlockSpec((_BM, N_CLASSES), lambda i: (i, 0)),
        out_shape=jax.ShapeDtypeStruct((NP, N_CLASSES), jnp.float32),
    )(agg2, h2p, dis, b2)


def kernel(x, edge_index, W1, b1, W2, b2):
    src = edge_index[0].astype(jnp.int32)
    dst = edge_index[1].astype(jnp.int32)

    npad = EP - N_EDGES
    pad_idx = N_NODES + (jnp.arange(npad, dtype=jnp.int32) % 8)
    src_p = jnp.concatenate([src, pad_idx]).reshape(EP // CHUNK, CHUNK)
    dst_p = jnp.concatenate([dst, pad_idx]).reshape(EP // CHUNK, CHUNK)

    zeros_128 = jnp.zeros((ROWS_PER_TILE, 128), jnp.float32)
    zeros_1 = jnp.zeros((NP,), jnp.float32)
    ones_c = jnp.ones((CHUNK,), jnp.float32)

    deg = _deg_kernel(dst_p, zeros_1, ones_c).reshape(NC, NP, 1)
    h1p, dis = _tc_k2(x, W1, deg)
    agg1 = _agg256_kernel(src_p, dst_p, h1p, zeros_128)
    h2p = _tc_k4(agg1, h1p, dis, b1.reshape(1, EMB_DIM), W2)
    agg2 = _agg40_kernel(src_p, dst_p, h2p, zeros_128)
    out = _tc_k6(agg2, h2p, dis, b2.reshape(1, N_CLASSES))
    return out[:N_NODES]


# speedup vs baseline: 1.0995x; 1.0712x over previous
import functools

import jax
import jax.numpy as jnp
from jax import lax
from jax.experimental import pallas as pl
from jax.experimental.pallas import tpu as pltpu
from jax.experimental.pallas import tpu_sc as plsc

N_NODES = 10000
D_FEAT = 256
EMB_DIM = 256
N_CLASSES = 40
N_EDGES = 160000

NC, NS = 2, 16
NW = NC * NS
CHUNK = 128
NP = 10112
EP = NW * 40 * CHUNK
ROWS_PER_TILE = NP // NS

_MESH = plsc.VectorSubcoreMesh(
    core_axis_name="c", subcore_axis_name="s", num_cores=NC, num_subcores=NS)


@functools.partial(
    pl.kernel,
    out_type=jax.ShapeDtypeStruct((NC, NP), jnp.float32),
    mesh=_MESH,
    scratch_types=[
        pltpu.VMEM((40, CHUNK), jnp.int32),
        pltpu.VMEM((CHUNK,), jnp.float32),
        pltpu.VMEM_SHARED((NP,), jnp.float32),
    ],
)
def _deg_kernel(dst_hbm, zer_hbm, one_hbm, out_hbm, dst_v, one_v, acc):
    c = lax.axis_index("c")
    s = lax.axis_index("s")
    w = c * NS + s
    pltpu.sync_copy(dst_hbm.at[pl.ds(w * 40, 40)], dst_v)
    pltpu.sync_copy(one_hbm, one_v)

    @pl.when(s == 0)
    def _():
        pltpu.sync_copy(zer_hbm, acc)

    plsc.subcore_barrier()

    def body(j, _):
        pltpu.sync_copy(one_v, acc.at[dst_v.at[j]], add=True)
        return ()

    lax.fori_loop(0, 40, body, (), unroll=False)
    plsc.subcore_barrier()

    @pl.when(s == 0)
    def _():
        pltpu.sync_copy(acc, out_hbm.at[c])


NBUF = 4
CH2 = 64
PASS_CHUNKS = 40


def _ring_pass(h_ref, src_v, dst_v, gbuf, acc, gsems, ssems):
    for b in range(NBUF):
        pltpu.async_copy(h_ref.at[src_v.at[b]], gbuf.at[b], gsems[b])

    def ring(r, _):
        for b in range(NBUF):
            cchunk = NBUF * r + b
            pltpu.make_async_copy(h_ref.at[src_v.at[0]], gbuf.at[b],
                                  gsems[b]).wait()
            pltpu.async_copy(gbuf.at[b], acc.at[dst_v.at[cchunk]], ssems[b],
                             add=True)
        for b in range(NBUF):
            @pl.when(r < PASS_CHUNKS // NBUF - 1)
            def _():
                pltpu.make_async_copy(gbuf.at[b], acc.at[dst_v.at[0]],
                                      ssems[b]).wait()
                pltpu.async_copy(h_ref.at[src_v.at[NBUF * (r + 1) + b]],
                                 gbuf.at[b], gsems[b])
        return ()

    lax.fori_loop(0, PASS_CHUNKS // NBUF, ring, (), unroll=False)
    for b in range(NBUF):
        pltpu.make_async_copy(gbuf.at[b], acc.at[dst_v.at[0]], ssems[b]).wait()


_AGG_SCRATCH = [
    pltpu.VMEM((PASS_CHUNKS, CH2), jnp.int32),
    pltpu.VMEM((PASS_CHUNKS, CH2), jnp.int32),
    pltpu.VMEM((NBUF, CH2, 128), jnp.float32),
    pltpu.VMEM_SHARED((NP, 128), jnp.float32),
    pltpu.SemaphoreType.DMA,
    pltpu.SemaphoreType.DMA,
    pltpu.SemaphoreType.DMA,
    pltpu.SemaphoreType.DMA,
    pltpu.SemaphoreType.DMA,
    pltpu.SemaphoreType.DMA,
    pltpu.SemaphoreType.DMA,
    pltpu.SemaphoreType.DMA,
]


@functools.partial(
    pl.kernel,
    out_type=jax.ShapeDtypeStruct((NC, NP, 128), jnp.float32),
    mesh=_MESH,
    scratch_types=list(_AGG_SCRATCH),
)
def _agg256_kernel(src_hbm, dst_hbm, h_hbm, zer_hbm, out_hbm,
                   src_v, dst_v, gbuf, acc, *sems):
    c = lax.axis_index("c")
    s = lax.axis_index("s")
    h_mine = h_hbm.at[c]
    gsems, ssems = sems[:NBUF], sems[NBUF:]
    pltpu.sync_copy(zer_hbm, acc.at[pl.ds(s * ROWS_PER_TILE, ROWS_PER_TILE)])
    plsc.subcore_barrier()
    for p in range(4):
        base = s * 160 + p * PASS_CHUNKS
        pltpu.sync_copy(src_hbm.at[pl.ds(base, PASS_CHUNKS)], src_v)
        pltpu.sync_copy(dst_hbm.at[pl.ds(base, PASS_CHUNKS)], dst_v)
        _ring_pass(h_mine, src_v, dst_v, gbuf, acc, gsems, ssems)
    plsc.subcore_barrier()
    pltpu.sync_copy(acc.at[pl.ds(s * ROWS_PER_TILE, ROWS_PER_TILE)],
                    out_hbm.at[c].at[pl.ds(s * ROWS_PER_TILE, ROWS_PER_TILE)])


@functools.partial(
    pl.kernel,
    out_type=jax.ShapeDtypeStruct((NC, NP, 128), jnp.float32),
    mesh=_MESH,
    scratch_types=list(_AGG_SCRATCH),
)
def _agg40_kernel(src_hbm, dst_hbm, h_hbm, zer_hbm, out_hbm,
                  src_v, dst_v, gbuf, acc, *sems):
    c = lax.axis_index("c")
    s = lax.axis_index("s")
    w = c * NS + s
    gsems, ssems = sems[:NBUF], sems[NBUF:]
    pltpu.sync_copy(zer_hbm,
                    acc.at[pl.ds(s * ROWS_PER_TILE, ROWS_PER_TILE)])
    plsc.subcore_barrier()
    for p in range(2):
        base = w * 80 + p * PASS_CHUNKS
        pltpu.sync_copy(src_hbm.at[pl.ds(base, PASS_CHUNKS)], src_v)
        pltpu.sync_copy(dst_hbm.at[pl.ds(base, PASS_CHUNKS)], dst_v)
        _ring_pass(h_hbm, src_v, dst_v, gbuf, acc, gsems, ssems)
    plsc.subcore_barrier()
    pltpu.sync_copy(acc.at[pl.ds(s * ROWS_PER_TILE, ROWS_PER_TILE)],
                    out_hbm.at[c].at[pl.ds(s * ROWS_PER_TILE, ROWS_PER_TILE)])


def _k2_body(x_ref, w1_ref, deg_ref, h1p_ref, dis_ref):
    deg = deg_ref[0] + deg_ref[1]
    dis = lax.rsqrt(deg + 1.0)
    h = jnp.dot(x_ref[...], w1_ref[...], preferred_element_type=jnp.float32)
    hp = h * dis
    h1p_ref[0] = hp[:, :128]
    h1p_ref[1] = hp[:, 128:]
    dis_ref[...] = dis


def _k4_body(agg_ref, h1p_ref, dis_ref, b1_ref, w2_ref, h2p_ref):
    dis = dis_ref[...]
    z0 = dis * (agg_ref[0] + h1p_ref[0])
    z1 = dis * (agg_ref[1] + h1p_ref[1])
    z = jnp.concatenate([z0, z1], axis=1) + b1_ref[...]
    z = jnp.maximum(z, 0.0)
    h2 = jnp.dot(z, w2_ref[...], preferred_element_type=jnp.float32)
    h2p_ref[...] = jnp.pad(h2 * dis, ((0, 0), (0, 128 - N_CLASSES)))


def _k6_body(agg_ref, h2p_ref, dis_ref, b2_ref, out_ref):
    s = agg_ref[0] + agg_ref[1] + h2p_ref[...]
    out_ref[...] = dis_ref[...] * s[:, :N_CLASSES] + b2_ref[...]


_BM = 2528


def _tc_k2(x, w1, deg):
    return pl.pallas_call(
        _k2_body,
        grid=(NP // _BM,),
        in_specs=[
            pl.BlockSpec((_BM, D_FEAT), lambda i: (i, 0)),
            pl.BlockSpec((D_FEAT, EMB_DIM), lambda i: (0, 0)),
            pl.BlockSpec((NC, _BM, 1), lambda i: (0, i, 0)),
        ],
        out_specs=[
            pl.BlockSpec((NC, _BM, 128), lambda i: (0, i, 0)),
            pl.BlockSpec((_BM, 1), lambda i: (i, 0)),
        ],
        out_shape=[
            jax.ShapeDtypeStruct((NC, NP, 128), jnp.float32),
            jax.ShapeDtypeStruct((NP, 1), jnp.float32),
        ],
    )(x, w1, deg)


def _tc_k4(agg1, h1p, dis, b1, w2):
    return pl.pallas_call(
        _k4_body,
        grid=(NP // _BM,),
        in_specs=[
            pl.BlockSpec((NC, _BM, 128), lambda i: (0, i, 0)),
            pl.BlockSpec((NC, _BM, 128), lambda i: (0, i, 0)),
            pl.BlockSpec((_BM, 1), lambda i: (i, 0)),
            pl.BlockSpec((1, EMB_DIM), lambda i: (0, 0)),
            pl.BlockSpec((EMB_DIM, N_CLASSES), lambda i: (0, 0)),
        ],
        out_specs=pl.BlockSpec((_BM, 128), lambda i: (i, 0)),
        out_shape=jax.ShapeDtypeStruct((NP, 128), jnp.float32),
    )(agg1, h1p, dis, b1, w2)


def _tc_k6(agg2, h2p, dis, b2):
    return pl.pallas_call(
        _k6_body,
        grid=(NP // _BM,),
        in_specs=[
            pl.BlockSpec((NC, _BM, 128), lambda i: (0, i, 0)),
            pl.BlockSpec((_BM, 128), lambda i: (i, 0)),
            pl.BlockSpec((_BM, 1), lambda i: (i, 0)),
            pl.BlockSpec((1, N_CLASSES), lambda i: (0, 0)),
        ],
        out_specs=pl.BlockSpec((_BM, N_CLASSES), lambda i: (i, 0)),
        out_shape=jax.ShapeDtypeStruct((NP, N_CLASSES), jnp.float32),
    )(agg2, h2p, dis, b2)


def kernel(x, edge_index, W1, b1, W2, b2):
    src = edge_index[0].astype(jnp.int32)
    dst = edge_index[1].astype(jnp.int32)

    npad = EP - N_EDGES
    pad_idx = N_NODES + (jnp.arange(npad, dtype=jnp.int32) % 8)
    src_p = jnp.concatenate([src, pad_idx]).reshape(EP // CHUNK, CHUNK)
    dst_p = jnp.concatenate([dst, pad_idx]).reshape(EP // CHUNK, CHUNK)
    src_p2 = src_p.reshape(EP // CH2, CH2)
    dst_p2 = dst_p.reshape(EP // CH2, CH2)

    zeros_128 = jnp.zeros((ROWS_PER_TILE, 128), jnp.float32)
    zeros_1 = jnp.zeros((NP,), jnp.float32)
    ones_c = jnp.ones((CHUNK,), jnp.float32)

    deg = _deg_kernel(dst_p, zeros_1, ones_c).reshape(NC, NP, 1)
    h1p, dis = _tc_k2(x, W1, deg)
    agg1 = _agg256_kernel(src_p2, dst_p2, h1p, zeros_128)
    h2p = _tc_k4(agg1, h1p, dis, b1.reshape(1, EMB_DIM), W2)
    agg2 = _agg40_kernel(src_p2, dst_p2, h2p, zeros_128)
    out = _tc_k6(agg2, h2p, dis, b2.reshape(1, N_CLASSES))
    return out[:N_NODES]
